# Initial kernel scaffold; baseline (speedup 1.0000x reference)
import jax
import jax.numpy as jnp
from jax.experimental import pallas as pl


def kernel(input_features, f_nodes, f_bonds, node2edge, edge2node, b2revb, fedges, a2a, W, b_lin, bias):
    raise NotImplementedError("write your pallas kernel here")


# speedup vs baseline: 1.0503x; 1.0503x over previous
import functools

import jax
import jax.numpy as jnp
from jax import lax
from jax.experimental import pallas as pl
from jax.experimental.pallas import tpu as pltpu
from jax.experimental.pallas import tpu_sc as plsc

_N, _E, _D, _K = 10000, 320000, 128, 32
_NC, _NS = 2, 16
_NW = _NC * _NS
_NPAD = 10240
_NPW = _NPAD // _NW
_B = 4
_NB = _NPW // _B
_RB = _B * _K
_C = _D // 16


def _support_matmul(x, w, b2d):
    def body(x_ref, w_ref, b_ref, o_ref):
        o_ref[...] = lax.dot_general(
            x_ref[...], w_ref[...], (((1,), (1,)), ((), ())),
            preferred_element_type=jnp.float32) + b_ref[...]

    return pl.pallas_call(
        body,
        grid=(10,),
        in_specs=[
            pl.BlockSpec((1000, _D), lambda i: (i, 0)),
            pl.BlockSpec((_D, _D), lambda i: (0, 0)),
            pl.BlockSpec((1, _D), lambda i: (0, 0)),
        ],
        out_specs=pl.BlockSpec((1000, _D), lambda i: (i, 0)),
        out_shape=jax.ShapeDtypeStruct((_N, _D), jnp.float32),
    )(x, w, b2d)


def _make_sc_kernel():
    mesh = plsc.VectorSubcoreMesh(core_axis_name="c", subcore_axis_name="s")

    @functools.partial(
        pl.kernel,
        out_type=jax.ShapeDtypeStruct((_NPAD, _D), jnp.float32),
        mesh=mesh,
        scratch_types=[
            pltpu.VMEM((_NPW * _K,), jnp.int32),
            pltpu.VMEM((_NPW * _K,), jnp.int32),
            pltpu.VMEM((_RB, _D), jnp.float32),
            pltpu.VMEM((_RB, _D), jnp.float32),
            pltpu.VMEM((_B, _D), jnp.float32),
            pltpu.VMEM((_D,), jnp.float32),
            pltpu.SemaphoreType.DMA,
            pltpu.SemaphoreType.DMA,
        ],
    )
    def sc(support_hbm, fedges_hbm, idxa_hbm, idxf_hbm, bias_hbm, out_hbm,
           idxa_v, idxf_v, arows, frows, outb, bias_v, sem_a, sem_f):
        cid = lax.axis_index("c")
        sid = lax.axis_index("s")
        wid = sid * _NC + cid
        base = wid * _NPW

        pltpu.sync_copy(idxa_hbm.at[pl.ds(base * _K, _NPW * _K)], idxa_v)
        pltpu.sync_copy(idxf_hbm.at[pl.ds(base * _K, _NPW * _K)], idxf_v)
        pltpu.sync_copy(bias_hbm, bias_v)

        def step(j, carry):
            a_cp = pltpu.make_async_copy(
                support_hbm.at[idxa_v.at[pl.ds(j * _RB, _RB)]], arows, sem_a)
            f_cp = pltpu.make_async_copy(
                fedges_hbm.at[idxf_v.at[pl.ds(j * _RB, _RB)]], frows, sem_f)
            a_cp.start()
            f_cp.start()
            a_cp.wait()
            f_cp.wait()
            for n in range(_B):
                acc = [None] * _C
                for k in range(_K):
                    r = n * _K + k
                    for c in range(_C):
                        prod = (arows[r, pl.ds(c * 16, 16)]
                                * frows[r, pl.ds(c * 16, 16)])
                        acc[c] = prod if acc[c] is None else acc[c] + prod
                for c in range(_C):
                    e = jnp.exp(acc[c] * 2.0)
                    t = 1.0 - 2.0 / (e + 1.0)
                    outb[n, pl.ds(c * 16, 16)] = t + bias_v[pl.ds(c * 16, 16)]
            pltpu.sync_copy(outb, out_hbm.at[pl.ds(base + j * _B, _B)])
            return carry

        lax.fori_loop(0, _NB, step, 0)

    return sc


_sc_kernel = _make_sc_kernel()


def kernel(input_features, f_nodes, f_bonds, node2edge, edge2node, b2revb,
           fedges, a2a, W, b_lin, bias):
    support = _support_matmul(input_features, W, b_lin.reshape(1, _D))
    pad = _NPAD - _N
    idxa = jnp.concatenate(
        [a2a, jnp.zeros((pad, _K), jnp.int32)], axis=0).reshape(-1)
    idxf = jnp.concatenate(
        [node2edge, jnp.zeros((pad, _K), jnp.int32)], axis=0).reshape(-1)
    out = _sc_kernel(support, fedges, idxa, idxf, bias)
    return out[:_N]


# speedup vs baseline: 1.6256x; 1.5478x over previous
import functools

import jax
import jax.numpy as jnp
from jax import lax
from jax.experimental import pallas as pl
from jax.experimental.pallas import tpu as pltpu
from jax.experimental.pallas import tpu_sc as plsc

_N, _E, _D, _K = 10000, 320000, 128, 32
_NC, _NS = 2, 16
_NW = _NC * _NS
_NPAD = 10240
_NPW = _NPAD // _NW
_B = 2
_NB = _NPW // _B
_RB = _B * _K
_C = _D // 16


def _support_matmul(x, w, b2d):
    def body(x_ref, w_ref, b_ref, o_ref):
        o_ref[...] = lax.dot_general(
            x_ref[...], w_ref[...], (((1,), (1,)), ((), ())),
            preferred_element_type=jnp.float32) + b_ref[...]

    return pl.pallas_call(
        body,
        grid=(10,),
        in_specs=[
            pl.BlockSpec((1000, _D), lambda i: (i, 0)),
            pl.BlockSpec((_D, _D), lambda i: (0, 0)),
            pl.BlockSpec((1, _D), lambda i: (0, 0)),
        ],
        out_specs=pl.BlockSpec((1000, _D), lambda i: (i, 0)),
        out_shape=jax.ShapeDtypeStruct((_N, _D), jnp.float32),
    )(x, w, b2d)


def _make_sc_kernel():
    mesh = plsc.VectorSubcoreMesh(core_axis_name="c", subcore_axis_name="s")

    @functools.partial(
        pl.kernel,
        out_type=jax.ShapeDtypeStruct((_NPAD, _D), jnp.float32),
        mesh=mesh,
        scratch_types=[
            pltpu.VMEM((_NPW * _K,), jnp.int32),
            pltpu.VMEM((_NPW * _K,), jnp.int32),
            pltpu.VMEM((2, _RB, _D), jnp.float32),
            pltpu.VMEM((2, _RB, _D), jnp.float32),
            pltpu.VMEM((2, _B, _D), jnp.float32),
            pltpu.VMEM((_D,), jnp.float32),
            [pltpu.SemaphoreType.DMA] * 2,
            [pltpu.SemaphoreType.DMA] * 2,
        ],
    )
    def sc(support_hbm, fedges_hbm, idxa_hbm, idxf_hbm, bias_hbm, out_hbm,
           idxa_v, idxf_v, arows, frows, outb, bias_v, sems_a, sems_f):
        cid = lax.axis_index("c")
        sid = lax.axis_index("s")
        wid = sid * _NC + cid
        base = wid * _NPW

        pltpu.sync_copy(idxa_hbm.at[pl.ds(base * _K, _NPW * _K)], idxa_v)
        pltpu.sync_copy(idxf_hbm.at[pl.ds(base * _K, _NPW * _K)], idxf_v)
        pltpu.sync_copy(bias_hbm, bias_v)

        def copies(j, slot):
            a_cp = pltpu.make_async_copy(
                support_hbm.at[idxa_v.at[pl.ds(j * _RB, _RB)]],
                arows.at[slot], sems_a[slot])
            f_cp = pltpu.make_async_copy(
                fedges_hbm.at[idxf_v.at[pl.ds(j * _RB, _RB)]],
                frows.at[slot], sems_f[slot])
            return a_cp, f_cp

        def fire(j, slot):
            a_cp, f_cp = copies(j, slot)
            a_cp.start()
            f_cp.start()

        def consume(j, slot):
            a_cp, f_cp = copies(j, slot)
            a_cp.wait()
            f_cp.wait()
            for n in range(_B):
                acc = [None] * _C
                for k in range(_K):
                    r = n * _K + k
                    for c in range(_C):
                        prod = (arows[slot, r, pl.ds(c * 16, 16)]
                                * frows[slot, r, pl.ds(c * 16, 16)])
                        acc[c] = prod if acc[c] is None else acc[c] + prod
                for c in range(_C):
                    e = jnp.exp(acc[c] * 2.0)
                    t = 1.0 - 2.0 / (e + 1.0)
                    outb[slot, n, pl.ds(c * 16, 16)] = t + bias_v[pl.ds(c * 16, 16)]
            pltpu.sync_copy(outb.at[slot], out_hbm.at[pl.ds(base + j * _B, _B)])

        fire(0, 0)
        fire(1, 1)

        def step(jj, carry):
            j = jj * 2
            for slot in range(2):
                consume(j + slot, slot)

                @pl.when(j + slot + 2 < _NB)
                def _():
                    fire(j + slot + 2, slot)

            return carry

        lax.fori_loop(0, _NB // 2, step, 0)

    return sc


_sc_kernel = _make_sc_kernel()


def kernel(input_features, f_nodes, f_bonds, node2edge, edge2node, b2revb,
           fedges, a2a, W, b_lin, bias):
    support = _support_matmul(input_features, W, b_lin.reshape(1, _D))
    pad = _NPAD - _N
    idxa = jnp.concatenate(
        [a2a, jnp.zeros((pad, _K), jnp.int32)], axis=0).reshape(-1)
    idxf = jnp.concatenate(
        [node2edge, jnp.zeros((pad, _K), jnp.int32)], axis=0).reshape(-1)
    out = _sc_kernel(support, fedges, idxa, idxf, bias)
    return out[:_N]
